# baseline (speedup 1.0000x reference)
import functools
import jax
import jax.numpy as jnp
from jax import lax
from jax.experimental import pallas as pl
from jax.experimental.pallas import tpu as pltpu
from jax.experimental.pallas import tpu_sc as plsc

NUM_BUCKETS = 1000000
DIM = 64
MODEL_DIM = 1024
BATCH = 4
SEQ = 8192

NC, NS, L = 2, 16, 16
NW = NC * NS
TOTAL = BATCH * SEQ
CHUNK = TOTAL // NW
HALF = CHUNK // 2
IDX_ROWS = CHUNK // 128
IDX_COLS = 128

PAD_BLK = 8192


def _pad_t_body(tt_ref, out_ref):
    xt = tt_ref[...].T.astype(jnp.bfloat16)
    bits = lax.bitcast_convert_type(xt, jnp.uint16)
    pairs = bits.reshape(PAD_BLK // 2, 2, DIM)
    lo = pairs[:, 0, :].astype(jnp.uint32)
    hi = pairs[:, 1, :].astype(jnp.uint32)
    packed = lax.bitcast_convert_type(lo | (hi << 16), jnp.int32)
    zeros = jnp.zeros((PAD_BLK // 2, DIM), jnp.int32)
    out_ref[...] = jnp.concatenate([packed, zeros], axis=1)


@jax.jit
def _pad_t(tt):
    return pl.pallas_call(
        _pad_t_body,
        grid=(pl.cdiv(NUM_BUCKETS, PAD_BLK),),
        in_specs=[pl.BlockSpec((DIM, PAD_BLK), lambda i: (0, i))],
        out_specs=pl.BlockSpec((PAD_BLK // 2, 2 * DIM), lambda i: (i, 0)),
        out_shape=jax.ShapeDtypeStruct(
            (NUM_BUCKETS // 2, 2 * DIM), jnp.int32
        ),
    )(tt)


def _sc_gather_body(ids_hbm, table_hbm, emb_hbm, par_hbm, ext_v, idx_v, par_v,
                    rows_v, sem):
    wid = lax.axis_index("s") * NC + lax.axis_index("c")
    base = wid * CHUNK

    pltpu.sync_copy(ids_hbm.at[pl.ds(base, CHUNK)], ext_v.at[pl.ds(L, CHUNK)])
    at_row_start = (base % SEQ) == 0

    @pl.when(at_row_start)
    def _():
        ext_v[pl.ds(0, L)] = jnp.zeros((L,), jnp.int32)

    @pl.when(jnp.logical_not(at_row_start))
    def _():
        pltpu.sync_copy(ids_hbm.at[pl.ds(base - L, L)], ext_v.at[pl.ds(0, L)])

    for j in range(IDX_ROWS):
        for t in range(IDX_COLS // L):
            i = j * (IDX_COLS // L) + t
            cur = ext_v[pl.ds(L + i * L, L)]
            prev = ext_v[pl.ds(L - 1 + i * L, L)]
            h = (prev * 31 + cur) % NUM_BUCKETS
            idx_v[j, pl.ds(t * L, L)] = h >> 1
            par_v[pl.ds(i * L, L)] = h & 1

    pltpu.sync_copy(par_v, par_hbm.at[pl.ds(base, CHUNK)])

    for half in range(2):
        copies = [
            pltpu.async_copy(
                table_hbm.at[idx_v.at[half * (IDX_ROWS // 2) + j]],
                rows_v.at[pl.ds(j * IDX_COLS, IDX_COLS)],
                sem,
            )
            for j in range(IDX_ROWS // 2)
        ]
        for c in copies:
            c.wait()
        pltpu.sync_copy(rows_v, emb_hbm.at[pl.ds(base + half * HALF, HALF)])


@jax.jit
def _sc_gather(ids_flat, table_pairs):
    mesh = plsc.VectorSubcoreMesh(
        core_axis_name="c", subcore_axis_name="s", num_cores=NC, num_subcores=NS
    )
    return pl.kernel(
        _sc_gather_body,
        out_type=(
            jax.ShapeDtypeStruct((TOTAL, 2 * DIM), jnp.int32),
            jax.ShapeDtypeStruct((TOTAL,), jnp.int32),
        ),
        mesh=mesh,
        scratch_types=[
            pltpu.VMEM((CHUNK + L,), jnp.int32),
            pltpu.VMEM((IDX_ROWS, IDX_COLS), jnp.int32),
            pltpu.VMEM((CHUNK,), jnp.int32),
            pltpu.VMEM((HALF, 2 * DIM), jnp.int32),
            pltpu.SemaphoreType.DMA,
        ],
    )(ids_flat, table_pairs)


ROWS_BLK = 2048


def _proj_body(emb_ref, par_ref, w_ref, out_ref):
    v = emb_ref[:, :DIM]
    sel = par_ref[0, :].reshape(ROWS_BLK, 1) == 1
    bits = jnp.where(sel, v & jnp.int32(-65536), v << 16)
    x = lax.bitcast_convert_type(bits, jnp.float32)
    out_ref[...] = lax.dot_general(
        x,
        w_ref[...],
        (((1,), (1,)), ((), ())),
        preferred_element_type=jnp.float32,
    )


@jax.jit
def _proj(emb, par, w):
    return pl.pallas_call(
        _proj_body,
        grid=(TOTAL // ROWS_BLK,),
        in_specs=[
            pl.BlockSpec((ROWS_BLK, 2 * DIM), lambda i: (i, 0)),
            pl.BlockSpec((1, ROWS_BLK), lambda i: (0, i)),
            pl.BlockSpec((MODEL_DIM, DIM), lambda i: (0, 0)),
        ],
        out_specs=pl.BlockSpec((ROWS_BLK, MODEL_DIM), lambda i: (i, 0)),
        out_shape=jax.ShapeDtypeStruct((TOTAL, MODEL_DIM), jnp.float32),
    )(emb, par.reshape(1, TOTAL), w)


def kernel(input_ids, embedding_weight, proj_weight):
    ids_flat = input_ids.reshape(-1)
    tt = embedding_weight.T
    table_pairs = _pad_t(tt)
    emb, par = _sc_gather(ids_flat, table_pairs)
    out = _proj(emb, par, proj_weight)
    return out.reshape(BATCH, SEQ, MODEL_DIM)


# speedup vs baseline: 1.9968x; 1.9968x over previous
import functools
import jax
import jax.numpy as jnp
from jax import lax
from jax.experimental import pallas as pl
from jax.experimental.pallas import tpu as pltpu
from jax.experimental.pallas import tpu_sc as plsc

NUM_BUCKETS = 1000000
DIM = 64
MODEL_DIM = 1024
BATCH = 4
SEQ = 8192

NC, NS, L = 2, 16, 16
NW = NC * NS
TOTAL = BATCH * SEQ
CHUNK = TOTAL // NW
HALF = CHUNK // 2
IDX_ROWS = CHUNK // 128
IDX_COLS = 128

PAD_BLK = 8192


NBLK = 123
PAIR_ROWS = NBLK * (PAD_BLK // 2)


def _pad_t_body(tt_ref, out_ref):
    t = tt_ref[...]
    out_ref[:, :DIM] = t[:, : PAD_BLK // 2].T
    out_ref[:, DIM:] = t[:, PAD_BLK // 2 :].T


@jax.jit
def _pad_t(tt):
    return pl.pallas_call(
        _pad_t_body,
        grid=(NBLK,),
        in_specs=[pl.BlockSpec((DIM, PAD_BLK), lambda i: (0, i))],
        out_specs=pl.BlockSpec((PAD_BLK // 2, 2 * DIM), lambda i: (i, 0)),
        out_shape=jax.ShapeDtypeStruct((PAIR_ROWS, 2 * DIM), jnp.float32),
    )(tt)


def _sc_gather_body(ids_hbm, table_hbm, emb_hbm, par_hbm, ext_v, idx_v, par_v,
                    rows_v, sem):
    wid = lax.axis_index("s") * NC + lax.axis_index("c")
    base = wid * CHUNK

    pltpu.sync_copy(ids_hbm.at[pl.ds(base, CHUNK)], ext_v.at[pl.ds(L, CHUNK)])
    at_row_start = (base % SEQ) == 0

    @pl.when(at_row_start)
    def _():
        ext_v[pl.ds(0, L)] = jnp.zeros((L,), jnp.int32)

    @pl.when(jnp.logical_not(at_row_start))
    def _():
        pltpu.sync_copy(ids_hbm.at[pl.ds(base - L, L)], ext_v.at[pl.ds(0, L)])

    for j in range(IDX_ROWS):
        for t in range(IDX_COLS // L):
            i = j * (IDX_COLS // L) + t
            cur = ext_v[pl.ds(L + i * L, L)]
            prev = ext_v[pl.ds(L - 1 + i * L, L)]
            h = (prev * 31 + cur) % NUM_BUCKETS
            idx_v[j, pl.ds(t * L, L)] = (h >> 13) * 4096 + (h & 4095)
            par_v[pl.ds(i * L, L)] = (h >> 12) & 1

    pltpu.sync_copy(par_v, par_hbm.at[pl.ds(base, CHUNK)])

    for half in range(2):
        copies = [
            pltpu.async_copy(
                table_hbm.at[idx_v.at[half * (IDX_ROWS // 2) + j]],
                rows_v.at[pl.ds(j * IDX_COLS, IDX_COLS)],
                sem,
            )
            for j in range(IDX_ROWS // 2)
        ]
        for c in copies:
            c.wait()
        pltpu.sync_copy(rows_v, emb_hbm.at[pl.ds(base + half * HALF, HALF)])


@jax.jit
def _sc_gather(ids_flat, table_pairs):
    mesh = plsc.VectorSubcoreMesh(
        core_axis_name="c", subcore_axis_name="s", num_cores=NC, num_subcores=NS
    )
    return pl.kernel(
        _sc_gather_body,
        out_type=(
            jax.ShapeDtypeStruct((TOTAL, 2 * DIM), jnp.float32),
            jax.ShapeDtypeStruct((TOTAL,), jnp.int32),
        ),
        mesh=mesh,
        scratch_types=[
            pltpu.VMEM((CHUNK + L,), jnp.int32),
            pltpu.VMEM((IDX_ROWS, IDX_COLS), jnp.int32),
            pltpu.VMEM((CHUNK,), jnp.int32),
            pltpu.VMEM((HALF, 2 * DIM), jnp.float32),
            pltpu.SemaphoreType.DMA,
        ],
    )(ids_flat, table_pairs)


ROWS_BLK = 2048


def _proj_body(emb_ref, par_ref, w2_ref, out_ref):
    par = par_ref[0, :].reshape(ROWS_BLK, 1)
    col = lax.broadcasted_iota(jnp.int32, (1, 2 * DIM), 1)
    keep = (col >= par * DIM) & (col < par * DIM + DIM)
    x = jnp.where(keep, emb_ref[...], 0.0)
    out_ref[...] = lax.dot_general(
        x,
        w2_ref[...],
        (((1,), (1,)), ((), ())),
        preferred_element_type=jnp.float32,
    )


@jax.jit
def _proj(emb, par, w):
    return pl.pallas_call(
        _proj_body,
        grid=(TOTAL // ROWS_BLK,),
        in_specs=[
            pl.BlockSpec((ROWS_BLK, 2 * DIM), lambda i: (i, 0)),
            pl.BlockSpec((1, ROWS_BLK), lambda i: (0, i)),
            pl.BlockSpec((MODEL_DIM, 2 * DIM), lambda i: (0, 0)),
        ],
        out_specs=pl.BlockSpec((ROWS_BLK, MODEL_DIM), lambda i: (i, 0)),
        out_shape=jax.ShapeDtypeStruct((TOTAL, MODEL_DIM), jnp.float32),
    )(emb, par.reshape(1, TOTAL), w)


def kernel(input_ids, embedding_weight, proj_weight):
    ids_flat = input_ids.reshape(-1)
    tt = embedding_weight.T
    table_pairs = _pad_t(tt)
    w2 = jnp.concatenate([proj_weight, proj_weight], axis=1)
    emb, par = _sc_gather(ids_flat, table_pairs)
    out = _proj(emb, par, w2)
    return out.reshape(BATCH, SEQ, MODEL_DIM)
